# baseline (speedup 1.0000x reference)
import functools

import jax
import jax.numpy as jnp
from jax import lax
from jax.experimental import pallas as pl
from jax.experimental.pallas import tpu as pltpu
from jax.experimental.pallas import tpu_sc as plsc

_NC = 2
_NS = 16
_NW = _NC * _NS
_ROWS = 16384
_H = 2048
_RPW = _ROWS // _NW
_CHUNK = 16
_NCH = _RPW // _CHUNK

_mesh = plsc.VectorSubcoreMesh(core_axis_name="c", subcore_axis_name="s")


@functools.partial(
    pl.kernel,
    mesh=_mesh,
    out_type=jax.ShapeDtypeStruct((_ROWS, _H), jnp.float32),
    scratch_types=[
        pltpu.VMEM((_CHUNK, _H), jnp.float32),
    ],
)
def _sc_copy(in_hbm, out_hbm, buf):
    wid = lax.axis_index("s") * _NC + lax.axis_index("c")
    base = wid * _RPW
    for i in range(_NCH):
        pltpu.sync_copy(in_hbm.at[pl.ds(base + i * _CHUNK, _CHUNK)], buf)
        pltpu.sync_copy(buf, out_hbm.at[pl.ds(base + i * _CHUNK, _CHUNK)])


def kernel(attention_mask, inputs_embeds):
    del attention_mask
    b, s, h = inputs_embeds.shape
    x = inputs_embeds.reshape(_ROWS, _H)
    return _sc_copy(x).reshape(b, s, h)


# speedup vs baseline: 1.6352x; 1.6352x over previous
import jax
import jax.numpy as jnp
from jax.experimental import pallas as pl

_BLOCK_ROWS = 1024


def _copy_kernel(in_ref, out_ref):
    out_ref[...] = in_ref[...]


def kernel(attention_mask, inputs_embeds):
    del attention_mask
    b, s, h = inputs_embeds.shape
    rows = b * s
    x = inputs_embeds.reshape(rows, h)
    out = pl.pallas_call(
        _copy_kernel,
        out_shape=jax.ShapeDtypeStruct((rows, h), x.dtype),
        grid=(rows // _BLOCK_ROWS,),
        in_specs=[pl.BlockSpec((_BLOCK_ROWS, h), lambda i: (i, 0))],
        out_specs=pl.BlockSpec((_BLOCK_ROWS, h), lambda i: (i, 0)),
    )(x)
    return out.reshape(b, s, h)
